# baseline (speedup 1.0000x reference)
import functools

import jax
import jax.numpy as jnp
from jax import lax
from jax.experimental import pallas as pl
from jax.experimental.pallas import tpu as pltpu
from jax.experimental.pallas import tpu_sc as plsc

_NC = 2
_NS = 16
_NW = _NC * _NS


@functools.lru_cache(maxsize=None)
def _sc_hist(n_edges: int, n_bins: int):
    body = n_edges // _NW
    lpt = body
    assert body * _NW == n_edges and body % 16 == 0 and body % 8 == 0
    mesh = plsc.VectorSubcoreMesh(core_axis_name="c", subcore_axis_name="s")

    @functools.partial(
        pl.kernel,
        mesh=mesh,
        out_type=[
            jax.ShapeDtypeStruct((_NC, n_bins), jnp.float32),
            jax.ShapeDtypeStruct((_NC, n_bins), jnp.float32),
        ],
        scratch_types=[
            pltpu.VMEM((lpt,), jnp.int32),
            pltpu.VMEM((lpt,), jnp.int32),
            pltpu.VMEM((lpt,), jnp.float32),
            pltpu.VMEM((lpt,), jnp.float32),
            pltpu.VMEM_SHARED((n_bins,), jnp.float32),
            pltpu.VMEM_SHARED((n_bins,), jnp.float32),
        ],
    )
    def k(ei_hbm, zeros_hbm, deg_out, cnt_out,
          src_v, dst_row, val_v, ones_v, deg_sh, cnt_sh):
        c = lax.axis_index("c")
        s = lax.axis_index("s")
        wid = c * _NS + s

        @pl.when(s == 0)
        def _zero():
            pltpu.sync_copy(zeros_hbm, deg_sh)
            pltpu.sync_copy(zeros_hbm, cnt_sh)

        pltpu.sync_copy(ei_hbm.at[pl.ds(wid * body, body)], src_v)
        pltpu.sync_copy(ei_hbm.at[pl.ds(n_edges + wid * body, body)], dst_row)

        one16 = jnp.full((16,), 1.0, jnp.float32)
        zero16 = jnp.zeros((16,), jnp.float32)

        def mkval(j, carry):
            sl = pl.ds(j * 16, 16)
            val_v[sl] = jnp.where(dst_row[sl] == 0, one16, zero16)
            ones_v[sl] = one16
            return carry

        lax.fori_loop(0, body // 16, mkval, 0, unroll=8)

        plsc.subcore_barrier()

        pltpu.sync_copy(ones_v, deg_sh.at[dst_row], add=True)
        pltpu.sync_copy(val_v, cnt_sh.at[src_v], add=True)

        plsc.subcore_barrier()

        @pl.when(s == 0)
        def _out():
            pltpu.sync_copy(deg_sh, deg_out.at[c])
            pltpu.sync_copy(cnt_sh, cnt_out.at[c])

    return k


def _tc_body(x_ref, degp_ref, cntp_ref, h_ref, we_ref, be_ref, wg_ref,
             bg_ref, wih_ref, bih_ref, whh_ref, bhh_ref, out_ref):
    f32 = jnp.float32
    x = x_ref[...]
    enc = jnp.maximum(
        jnp.dot(x, we_ref[...], preferred_element_type=f32) + be_ref[...],
        0.0)

    deg = degp_ref[0:1, :] + degp_ref[1:2, :] + 1.0
    dinv = lax.rsqrt(deg)
    cnt = cntp_ref[0:1, :] + cntp_ref[1:2, :]
    wrow = cnt * dinv

    vsum = jnp.dot(wrow, enc, preferred_element_type=f32)
    dinv0 = dinv[0:1, 0:1]
    v = dinv0 * vsum + (dinv0 * dinv0) * enc[0:1, :]

    agg0 = jnp.dot(v, wg_ref[...], preferred_element_type=f32)
    g = jnp.maximum(agg0 + bg_ref[...], 0.0)

    gi = jnp.dot(g, wih_ref[...], preferred_element_type=f32) + bih_ref[...]
    h0 = h_ref[...]
    gh = jnp.dot(h0, whh_ref[...], preferred_element_type=f32) + bhh_ref[...]
    hdim = h0.shape[1]
    i_r, i_z, i_n = (gi[:, 0:hdim], gi[:, hdim:2 * hdim], gi[:, 2 * hdim:])
    h_r, h_z, h_n = (gh[:, 0:hdim], gh[:, hdim:2 * hdim], gh[:, 2 * hdim:])
    r = jax.nn.sigmoid(i_r + h_r)
    z = jax.nn.sigmoid(i_z + h_z)
    n = jnp.tanh(i_n + r * h_n)
    out_ref[...] = (1.0 - z) * n + z * h0


def kernel(node_features, edge_index, edge_attr, hidden_state,
           W_enc, b_enc, W_gcn, b_gcn, w_ih, b_ih, w_hh, b_hh):
    del edge_attr
    n_nodes, d = node_features.shape
    e = edge_index.shape[1]

    zeros = jnp.zeros((n_nodes,), jnp.float32)
    ei_flat = edge_index.reshape(2 * e)
    degp, cntp = _sc_hist(e, n_nodes)(ei_flat, zeros)

    return pl.pallas_call(
        _tc_body,
        out_shape=jax.ShapeDtypeStruct((1, hidden_state.shape[1]),
                                       jnp.float32),
    )(node_features, degp, cntp, hidden_state,
      W_enc, b_enc.reshape(1, d), W_gcn, b_gcn.reshape(1, d),
      w_ih, b_ih.reshape(1, -1), w_hh, b_hh.reshape(1, -1))


# speedup vs baseline: 1.0639x; 1.0639x over previous
import functools

import jax
import jax.numpy as jnp
from jax import lax
from jax.experimental import pallas as pl
from jax.experimental.pallas import tpu as pltpu
from jax.experimental.pallas import tpu_sc as plsc

_NC = 2
_NS = 16
_NW = _NC * _NS


@functools.lru_cache(maxsize=None)
def _sc_hist(n_edges: int, n_bins: int):
    body = n_edges // _NW
    lpt = body
    assert body * _NW == n_edges and body % 16 == 0 and body % 8 == 0
    mesh = plsc.VectorSubcoreMesh(core_axis_name="c", subcore_axis_name="s")

    @functools.partial(
        pl.kernel,
        mesh=mesh,
        out_type=[
            jax.ShapeDtypeStruct((_NC, n_bins), jnp.float32),
            jax.ShapeDtypeStruct((_NC, n_bins), jnp.float32),
        ],
        scratch_types=[
            pltpu.VMEM((lpt,), jnp.int32),
            pltpu.VMEM((lpt,), jnp.int32),
            pltpu.VMEM((lpt,), jnp.float32),
            pltpu.VMEM((lpt,), jnp.float32),
            pltpu.VMEM_SHARED((n_bins,), jnp.float32),
            pltpu.VMEM_SHARED((n_bins,), jnp.float32),
        ],
    )
    def k(ei_hbm, zeros_hbm, deg_out, cnt_out,
          src_v, dst_row, val_v, ones_v, deg_sh, cnt_sh):
        c = lax.axis_index("c")
        s = lax.axis_index("s")
        wid = c * _NS + s

        @pl.when(s == 0)
        def _zero():
            pltpu.sync_copy(zeros_hbm, deg_sh)
            pltpu.sync_copy(zeros_hbm, cnt_sh)

        pltpu.sync_copy(ei_hbm.at[pl.ds(wid * body, body)], src_v)
        pltpu.sync_copy(ei_hbm.at[pl.ds(n_edges + wid * body, body)], dst_row)

        one16 = jnp.full((16,), 1.0, jnp.float32)
        zero16 = jnp.zeros((16,), jnp.float32)

        def mkval(j, carry):
            sl = pl.ds(j * 16, 16)
            val_v[sl] = jnp.where(dst_row[sl] == 0, one16, zero16)
            ones_v[sl] = one16
            return carry

        lax.fori_loop(0, body // 16, mkval, 0)

        plsc.subcore_barrier()

        pltpu.sync_copy(ones_v, deg_sh.at[dst_row], add=True)
        pltpu.sync_copy(val_v, cnt_sh.at[src_v], add=True)

        plsc.subcore_barrier()

        @pl.when(s == 0)
        def _out():
            pltpu.sync_copy(deg_sh, deg_out.at[c])
            pltpu.sync_copy(cnt_sh, cnt_out.at[c])

    return k


def _enc_body(x_ref, we_ref, be_ref, enc_ref):
    enc = jnp.maximum(
        jnp.dot(x_ref[...], we_ref[...],
                preferred_element_type=jnp.float32) + be_ref[...],
        0.0)
    enc_ref[...] = enc.astype(jnp.bfloat16)


def _tc_body(enc_ref, degp_ref, cntp_ref, h_ref, wg_ref,
             bg_ref, wih_ref, bih_ref, whh_ref, bhh_ref, out_ref):
    f32 = jnp.float32
    enc = enc_ref[...].astype(f32)

    deg = degp_ref[0:1, :] + degp_ref[1:2, :] + 1.0
    dinv = lax.rsqrt(deg)
    cnt = cntp_ref[0:1, :] + cntp_ref[1:2, :]
    wrow = cnt * dinv

    vsum = jnp.dot(wrow, enc, preferred_element_type=f32)
    dinv0 = dinv[0:1, 0:1]
    v = dinv0 * vsum + (dinv0 * dinv0) * enc[0:1, :]

    agg0 = jnp.dot(v, wg_ref[...], preferred_element_type=f32)
    g = jnp.maximum(agg0 + bg_ref[...], 0.0)

    gi = jnp.dot(g, wih_ref[...], preferred_element_type=f32) + bih_ref[...]
    h0 = h_ref[...]
    gh = jnp.dot(h0, whh_ref[...], preferred_element_type=f32) + bhh_ref[...]
    hdim = h0.shape[1]
    i_r, i_z, i_n = (gi[:, 0:hdim], gi[:, hdim:2 * hdim], gi[:, 2 * hdim:])
    h_r, h_z, h_n = (gh[:, 0:hdim], gh[:, hdim:2 * hdim], gh[:, 2 * hdim:])
    r = jax.nn.sigmoid(i_r + h_r)
    z = jax.nn.sigmoid(i_z + h_z)
    n = jnp.tanh(i_n + r * h_n)
    out_ref[...] = (1.0 - z) * n + z * h0


def kernel(node_features, edge_index, edge_attr, hidden_state,
           W_enc, b_enc, W_gcn, b_gcn, w_ih, b_ih, w_hh, b_hh):
    del edge_attr
    n_nodes, d = node_features.shape
    e = edge_index.shape[1]

    zeros = jnp.zeros((n_nodes,), jnp.float32)
    ei_flat = edge_index.reshape(2 * e)
    degp, cntp = _sc_hist(e, n_nodes)(ei_flat, zeros)

    encb = pl.pallas_call(
        _enc_body,
        out_shape=jax.ShapeDtypeStruct((n_nodes, d), jnp.bfloat16),
    )(node_features, W_enc, b_enc.reshape(1, d))

    return pl.pallas_call(
        _tc_body,
        out_shape=jax.ShapeDtypeStruct((1, hidden_state.shape[1]),
                                       jnp.float32),
    )(encb, degp, cntp, hidden_state,
      W_gcn, b_gcn.reshape(1, d),
      w_ih, b_ih.reshape(1, -1), w_hh, b_hh.reshape(1, -1))


# speedup vs baseline: 1.0980x; 1.0320x over previous
import functools

import jax
import jax.numpy as jnp
from jax import lax
from jax.experimental import pallas as pl
from jax.experimental.pallas import tpu as pltpu
from jax.experimental.pallas import tpu_sc as plsc

_NC = 2
_NS = 16
_NW = _NC * _NS


@functools.lru_cache(maxsize=None)
def _sc_hist(n_edges: int, n_bins: int):
    body = n_edges // _NW
    lpt = body
    assert body * _NW == n_edges and body % 16 == 0 and body % 8 == 0
    mesh = plsc.VectorSubcoreMesh(core_axis_name="c", subcore_axis_name="s")

    @functools.partial(
        pl.kernel,
        mesh=mesh,
        out_type=[
            jax.ShapeDtypeStruct((_NC, n_bins), jnp.float32),
            jax.ShapeDtypeStruct((_NC, n_bins), jnp.float32),
        ],
        scratch_types=[
            pltpu.VMEM((lpt,), jnp.int32),
            pltpu.VMEM((lpt,), jnp.int32),
            pltpu.VMEM((lpt,), jnp.float32),
            pltpu.VMEM((lpt,), jnp.float32),
            pltpu.VMEM_SHARED((n_bins,), jnp.float32),
            pltpu.VMEM_SHARED((n_bins,), jnp.float32),
            pltpu.SemaphoreType.DMA,
            pltpu.SemaphoreType.DMA,
        ],
    )
    def k(ei_hbm, zeros_hbm, deg_out, cnt_out,
          src_v, dst_row, val_v, ones_v, deg_sh, cnt_sh, sem_in, sem_sc):
        c = lax.axis_index("c")
        s = lax.axis_index("s")
        wid = c * _NS + s

        hs = pltpu.async_copy(ei_hbm.at[pl.ds(wid * body, body)],
                              src_v, sem_in)
        hd = pltpu.async_copy(ei_hbm.at[pl.ds(n_edges + wid * body, body)],
                              dst_row, sem_in)

        @pl.when(s == 0)
        def _zero():
            pltpu.sync_copy(zeros_hbm, deg_sh)
            pltpu.sync_copy(zeros_hbm, cnt_sh)

        one16 = jnp.full((16,), 1.0, jnp.float32)
        zero16 = jnp.zeros((16,), jnp.float32)

        def mkones(j, carry):
            ones_v[pl.ds(j * 16, 16)] = one16
            return carry

        lax.fori_loop(0, body // 16, mkones, 0)

        hs.wait()
        hd.wait()
        plsc.subcore_barrier()

        hdeg = pltpu.async_copy(ones_v, deg_sh.at[dst_row], sem_sc, add=True)

        def mkval(j, carry):
            sl = pl.ds(j * 16, 16)
            val_v[sl] = jnp.where(dst_row[sl] == 0, one16, zero16)
            return carry

        lax.fori_loop(0, body // 16, mkval, 0)

        pltpu.sync_copy(val_v, cnt_sh.at[src_v], add=True)
        hdeg.wait()

        plsc.subcore_barrier()

        @pl.when(s == 0)
        def _out():
            pltpu.sync_copy(deg_sh, deg_out.at[c])
            pltpu.sync_copy(cnt_sh, cnt_out.at[c])

    return k


def _enc_body(x_ref, we_ref, be_ref, enc_ref):
    enc = jnp.maximum(
        jnp.dot(x_ref[...], we_ref[...],
                preferred_element_type=jnp.float32) + be_ref[...],
        0.0)
    enc_ref[...] = enc.astype(jnp.bfloat16)


def _tc_body(enc_ref, degp_ref, cntp_ref, h_ref, wg_ref,
             bg_ref, wih_ref, bih_ref, whh_ref, bhh_ref, out_ref):
    f32 = jnp.float32
    enc = enc_ref[...].astype(f32)

    deg = degp_ref[0:1, :] + degp_ref[1:2, :] + 1.0
    dinv = lax.rsqrt(deg)
    cnt = cntp_ref[0:1, :] + cntp_ref[1:2, :]
    wrow = cnt * dinv

    vsum = jnp.dot(wrow, enc, preferred_element_type=f32)
    dinv0 = dinv[0:1, 0:1]
    v = dinv0 * vsum + (dinv0 * dinv0) * enc[0:1, :]

    agg0 = jnp.dot(v, wg_ref[...], preferred_element_type=f32)
    g = jnp.maximum(agg0 + bg_ref[...], 0.0)

    gi = jnp.dot(g, wih_ref[...], preferred_element_type=f32) + bih_ref[...]
    h0 = h_ref[...]
    gh = jnp.dot(h0, whh_ref[...], preferred_element_type=f32) + bhh_ref[...]
    hdim = h0.shape[1]
    i_r, i_z, i_n = (gi[:, 0:hdim], gi[:, hdim:2 * hdim], gi[:, 2 * hdim:])
    h_r, h_z, h_n = (gh[:, 0:hdim], gh[:, hdim:2 * hdim], gh[:, 2 * hdim:])
    r = jax.nn.sigmoid(i_r + h_r)
    z = jax.nn.sigmoid(i_z + h_z)
    n = jnp.tanh(i_n + r * h_n)
    out_ref[...] = (1.0 - z) * n + z * h0


def kernel(node_features, edge_index, edge_attr, hidden_state,
           W_enc, b_enc, W_gcn, b_gcn, w_ih, b_ih, w_hh, b_hh):
    del edge_attr
    n_nodes, d = node_features.shape
    e = edge_index.shape[1]

    zeros = jnp.zeros((n_nodes,), jnp.float32)
    ei_flat = edge_index.reshape(2 * e)
    degp, cntp = _sc_hist(e, n_nodes)(ei_flat, zeros)

    encb = pl.pallas_call(
        _enc_body,
        out_shape=jax.ShapeDtypeStruct((n_nodes, d), jnp.bfloat16),
    )(node_features, W_enc, b_enc.reshape(1, d))

    return pl.pallas_call(
        _tc_body,
        out_shape=jax.ShapeDtypeStruct((1, hidden_state.shape[1]),
                                       jnp.float32),
    )(encb, degp, cntp, hidden_state,
      W_gcn, b_gcn.reshape(1, d),
      w_ih, b_ih.reshape(1, -1), w_hh, b_hh.reshape(1, -1))
